# baseline (speedup 1.0000x reference)
import functools

import jax
import jax.numpy as jnp
from jax import lax
from jax.experimental import pallas as pl
from jax.experimental.pallas import tpu as pltpu
from jax.experimental.pallas import tpu_sc as plsc

N = 10000
E = 160000
D = 256
L = 16
L2 = 32
NW = 32
NP = 320
CH = 4000
NCH = E // CH
NG = CH // L
B = 32
SLOTS = 8
RB = 16
NEG = float("-inf")


def _gcn_body(x_hbm, xp_hbm, src_hbm, dst_hbm, out_hbm,
              acc, srcb0, dstb0, srcb1, dstb1, gsrc0, gdst0, gsrc1, gdst1,
              rowb, inb, outb,
              touched, esem0, esem1, gsem, lsem):
  wid = lax.axis_index("s") * 2 + lax.axis_index("c")
  n0 = wid * NP
  negi = jnp.full((L,), -8323200, jnp.int32)

  def init_acc(i, _):
    for j in range(D // L2):
      acc[i, pl.ds(j * L, L)] = negi
    return 0
  lax.fori_loop(0, NP + 1, init_acc, 0)

  def init_touched(i, _):
    touched[i] = 0
    return 0
  lax.fori_loop(0, NP + 1, init_touched, 0)

  zero16 = jnp.zeros((L,), jnp.int32)
  trash16 = jnp.full((L,), NP, jnp.int32)

  def fire_edges(c, sb, db, sem):
    pltpu.async_copy(src_hbm.at[pl.ds(c * CH, CH)], sb, sem)
    pltpu.async_copy(dst_hbm.at[pl.ds(c * CH, CH)], db, sem)

  def wait_edges(c, sb, db, sem):
    pltpu.make_async_copy(src_hbm.at[pl.ds(c * CH, CH)], sb, sem).wait()
    pltpu.make_async_copy(dst_hbm.at[pl.ds(c * CH, CH)], db, sem).wait()

  def load_row(r, j):
    return plsc.bitcast(rowb[r, pl.ds(j * L, L)], jnp.bfloat16)

  npv = jnp.full((L,), NP, jnp.uint32)

  def filter_chunk(sb, db, gs, gd):
    def filt1(g, cntv):
      d0 = db[pl.ds(g * L, L)]
      s0 = sb[pl.ds(g * L, L)]
      dl0 = d0 - n0
      m0 = plsc.bitcast(dl0, jnp.uint32) < npv
      csum0 = plsc.cumsum(jnp.where(m0, 1, 0))
      pc0v = plsc.all_reduce_population_count(m0)
      pos0 = cntv + csum0 - 1
      plsc.store_scatter(gs, [pos0], s0, mask=m0)
      plsc.store_scatter(gd, [pos0], dl0, mask=m0)
      return cntv + pc0v

    def filt2(g2, cntv):
      cntv = filt1(g2 * 2, cntv)
      return filt1(g2 * 2 + 1, cntv)

    with jax.named_scope("filt"):
      cntv = lax.fori_loop(0, NG // 2, filt2, jnp.zeros((L,), jnp.int32))
      if NG % 2:
        cntv = filt1(NG - 1, cntv)
    cnt = cntv[0]

    gs[pl.ds(cnt, L)] = zero16
    gs[pl.ds(cnt + L, L)] = zero16
    gd[pl.ds(cnt, L)] = trash16
    gd[pl.ds(cnt + L, L)] = trash16
    return (cnt + B - 1) // B

  def fire_gather(gs, b, slot):
    pltpu.async_copy(xp_hbm.at[gs.at[pl.ds(b * B, B)]],
                     rowb.at[pl.ds(slot * B, B)], gsem)

  def wait_gather(gs, b, slot):
    pltpu.make_async_copy(xp_hbm.at[gs.at[pl.ds(b * B, B)]],
                          rowb.at[pl.ds(slot * B, B)], gsem).wait()

  def prime(nb, gs):
    for k in range(SLOTS - 1):
      @pl.when(k < nb)
      def _(k=k):
        fire_gather(gs, k, k)

  def drain(nb, gs, gd):
   with jax.named_scope("gap"):
    def body(b, _):
      slot = b & (SLOTS - 1)
      wait_gather(gs, b, slot)

      @pl.when(b + SLOTS - 1 < nb)
      def _():
        fire_gather(gs, b + SLOTS - 1, (b + SLOTS - 1) & (SLOTS - 1))

      for h in range(B // L):
        dlv = gd[pl.ds(b * B + h * L, L)]
        for i in range(L):
          dl = dlv[i]
          touched[dl] = 1
          r = slot * B + h * L + i
          avs = [plsc.bitcast(acc[dl, pl.ds(j * L, L)], jnp.bfloat16)
                 for j in range(D // L2)]
          rvs = [load_row(r, j) for j in range(D // L2)]
          for j in range(D // L2):
            mx = jnp.maximum(avs[j], rvs[j])
            acc[dl, pl.ds(j * L, L)] = plsc.bitcast(mx, jnp.int32)
      return 0

    lax.fori_loop(0, nb, body, 0)

  fire_edges(0, srcb0, dstb0, esem0)
  wait_edges(0, srcb0, dstb0, esem0)
  nb0 = filter_chunk(srcb0, dstb0, gsrc0, gdst0)
  fire_edges(1, srcb1, dstb1, esem1)
  prime(nb0, gsrc0)

  def cpair(cc, nb_prev):
    a = 2 * cc + 1
    wait_edges(a, srcb1, dstb1, esem1)
    nb_a = filter_chunk(srcb1, dstb1, gsrc1, gdst1)
    fire_edges(a + 1, srcb0, dstb0, esem0)
    drain(nb_prev, gsrc0, gdst0)
    prime(nb_a, gsrc1)

    b = a + 1
    wait_edges(b, srcb0, dstb0, esem0)
    nb_b = filter_chunk(srcb0, dstb0, gsrc0, gdst0)

    @pl.when(b + 1 < NCH)
    def _():
      fire_edges(b + 1, srcb1, dstb1, esem1)
    drain(nb_a, gsrc1, gdst1)
    prime(nb_b, gsrc0)
    return nb_b

  nb_last = lax.fori_loop(0, (NCH - 2) // 2, cpair, nb0)

  wait_edges(NCH - 1, srcb1, dstb1, esem1)
  nb_t = filter_chunk(srcb1, dstb1, gsrc1, gdst1)
  drain(nb_last, gsrc0, gdst0)
  prime(nb_t, gsrc1)
  drain(nb_t, gsrc1, gdst1)

  himask = jnp.full((L,), -65536, jnp.int32)

  def write_body(b, _):
   with jax.named_scope("wout"):
    @pl.when(n0 + b * RB < N)
    def _():
      pltpu.async_copy(
          x_hbm.at[pl.ds(n0 + b * RB, RB)],
          out_hbm.at[pl.ds(n0 + b * RB, RB), pl.ds(0, D)], lsem)
      pltpu.sync_copy(x_hbm.at[pl.ds(n0 + b * RB, RB)], inb)
      for r in range(RB):
        t = touched[b * RB + r]
        tv = lax.broadcast_in_dim(t, (L,), ()) > 0
        for j in range(D // L2):
          w = acc[b * RB + r, pl.ds(j * L, L)]
          fa = plsc.bitcast(w << 16, jnp.float32)
          fb = plsc.bitcast(w & himask, jnp.float32)
          iva = inb[r, pl.ds(j * L2, L)]
          ivb = inb[r, pl.ds(j * L2 + L, L)]
          outb[r, pl.ds(j * L2, L)] = jnp.where(tv, fa, iva)
          outb[r, pl.ds(j * L2 + L, L)] = jnp.where(tv, fb, ivb)
      pltpu.sync_copy(outb, out_hbm.at[pl.ds(n0 + b * RB, RB), pl.ds(D, D)])
    return 0

  lax.fori_loop(0, NP // RB, write_body, 0)

  def drain_left(b, _):
    @pl.when(n0 + b * RB < N)
    def _():
      pltpu.make_async_copy(
          x_hbm.at[pl.ds(n0 + b * RB, RB)],
          out_hbm.at[pl.ds(n0 + b * RB, RB), pl.ds(0, D)], lsem).wait()
    return 0

  lax.fori_loop(0, NP // RB, drain_left, 0)


@functools.partial(jax.jit, donate_argnums=())
def _gcn(x, src, dst):
  xp = (x.reshape(N, D // L2, 2, L)
        .transpose(0, 1, 3, 2)
        .reshape(N, D)
        .astype(jnp.bfloat16))
  xp = jax.lax.bitcast_convert_type(xp.reshape(N, D // 2, 2), jnp.int32)
  mesh = plsc.VectorSubcoreMesh(core_axis_name="c", subcore_axis_name="s")
  run = pl.kernel(
      _gcn_body,
      compiler_params=pltpu.CompilerParams(needs_layout_passes=False),
      out_type=jax.ShapeDtypeStruct((N, 2 * D), jnp.float32),
      mesh=mesh,
      scratch_types=[
          pltpu.VMEM((NP + 1, D // 2), jnp.int32),
          pltpu.VMEM((CH,), jnp.int32),
          pltpu.VMEM((CH,), jnp.int32),
          pltpu.VMEM((CH,), jnp.int32),
          pltpu.VMEM((CH,), jnp.int32),
          pltpu.VMEM((CH + B,), jnp.int32),
          pltpu.VMEM((CH + B,), jnp.int32),
          pltpu.VMEM((CH + B,), jnp.int32),
          pltpu.VMEM((CH + B,), jnp.int32),
          pltpu.VMEM((SLOTS * B, D // 2), jnp.int32),
          pltpu.VMEM((RB, D), jnp.float32),
          pltpu.VMEM((RB, D), jnp.float32),
          pltpu.SMEM((NP + 1,), jnp.int32),
          pltpu.SemaphoreType.DMA,
          pltpu.SemaphoreType.DMA,
          pltpu.SemaphoreType.DMA,
          pltpu.SemaphoreType.DMA,
      ],
  )
  return run(x, xp, src, dst)


def kernel(inputs, edge_index):
  return _gcn(inputs, edge_index[0], edge_index[1])


# speedup vs baseline: 1.7041x; 1.7041x over previous
import functools

import jax
import jax.numpy as jnp
from jax import lax
from jax.experimental import pallas as pl
from jax.experimental.pallas import tpu as pltpu
from jax.experimental.pallas import tpu_sc as plsc

N = 10000
E = 160000
D = 256
L = 16
L2 = 32
NW = 32
NP = 320
CH = 16000
NCH = E // CH
NG = CH // L
B = 32
NBUF = 3
RB = 16
NEG = float("-inf")


def _gcn_body(x_hbm, xp_hbm, src_hbm, dst_hbm, out_hbm,
              acc, srcb0, dstb0, gsrc, gdst,
              rowb0, rowb1, rowb2, inb, outb,
              touched, esem0, gsem0, gsem1, gsem2, lsem):
  wid = lax.axis_index("s") * 2 + lax.axis_index("c")
  n0 = wid * NP
  rowbs = [rowb0, rowb1, rowb2]
  gsems = [gsem0, gsem1, gsem2]

  negi = jnp.full((L,), -8323200, jnp.int32)

  def init_acc(i, _):
    for j in range(D // L2):
      acc[i, pl.ds(j * L, L)] = negi
    return 0
  lax.fori_loop(0, NP + 1, init_acc, 0)

  def init_touched(i, _):
    touched[i] = 0
    return 0
  lax.fori_loop(0, NP + 1, init_touched, 0)

  zero16 = jnp.zeros((L,), jnp.int32)
  trash16 = jnp.full((L,), NP, jnp.int32)

  def fire_edges(c, sb, db, sem):
    pltpu.async_copy(src_hbm.at[pl.ds(c * CH, CH)], sb, sem)
    pltpu.async_copy(dst_hbm.at[pl.ds(c * CH, CH)], db, sem)

  def wait_edges(c, sb, db, sem):
    pltpu.make_async_copy(src_hbm.at[pl.ds(c * CH, CH)], sb, sem).wait()
    pltpu.make_async_copy(dst_hbm.at[pl.ds(c * CH, CH)], db, sem).wait()

  def fire_gather(b, rb, sem):
    pltpu.async_copy(xp_hbm.at[gsrc.at[pl.ds(b * B, B)]], rb, sem)

  def wait_gather(b, rb, sem):
    pltpu.make_async_copy(xp_hbm.at[gsrc.at[pl.ds(b * B, B)]], rb, sem).wait()

  def load_row(rb, r, j):
    return plsc.bitcast(rb[r, pl.ds(j * L, L)], jnp.bfloat16)

  def process_chunk(c, sb, db):
    npv = jnp.full((L,), NP, jnp.uint32)

    def filt1(g, cntv):
      d0 = db[pl.ds(g * L, L)]
      s0 = sb[pl.ds(g * L, L)]
      dl0 = d0 - n0
      m0 = plsc.bitcast(dl0, jnp.uint32) < npv
      csum0 = plsc.cumsum(jnp.where(m0, 1, 0))
      pc0v = plsc.all_reduce_population_count(m0)
      pos0 = cntv + csum0 - 1
      plsc.store_scatter(gsrc, [pos0], s0, mask=m0)
      plsc.store_scatter(gdst, [pos0], dl0, mask=m0)
      return cntv + pc0v

    def filt2(g2, cntv):
      cntv = filt1(g2 * 2, cntv)
      return filt1(g2 * 2 + 1, cntv)

    with jax.named_scope("filt"):
      cntv = lax.fori_loop(0, NG // 2, filt2, jnp.zeros((L,), jnp.int32))
      if NG % 2:
        cntv = filt1(NG - 1, cntv)
    cnt = cntv[0]

    gsrc[pl.ds(cnt, L)] = zero16
    gsrc[pl.ds(cnt + L, L)] = zero16
    gdst[pl.ds(cnt, L)] = trash16
    gdst[pl.ds(cnt + L, L)] = trash16

    nb = (cnt + B - 1) // B

    @pl.when(c + 1 < NCH)
    def _():
      fire_edges(c + 1, sb, db, esem0)

    def apply(b, rb):
      for h in range(B // L):
        dlv = gdst[pl.ds(b * B + h * L, L)]
        for i in range(L):
          dl = dlv[i]
          touched[dl] = 1
          r = h * L + i
          avs = [plsc.bitcast(acc[dl, pl.ds(j * L, L)], jnp.bfloat16)
                 for j in range(D // L2)]
          rvs = [load_row(rb, r, j) for j in range(D // L2)]
          for j in range(D // L2):
            mx = jnp.maximum(avs[j], rvs[j])
            acc[dl, pl.ds(j * L, L)] = plsc.bitcast(mx, jnp.int32)

    @pl.when(nb > 0)
    def _():
     with jax.named_scope("gap"):
      for k in range(NBUF - 1):
        @pl.when(k < nb)
        def _(k=k):
          fire_gather(k, rowbs[k], gsems[k])

      def gtri(bb, _):
        for k in range(NBUF):
          b = bb * NBUF + k

          @pl.when(b < nb)
          def _(b=b, k=k):
            wait_gather(b, rowbs[k], gsems[k])
            nxt = b + NBUF - 1
            kn = (k + NBUF - 1) % NBUF

            @pl.when(nxt < nb)
            def _(nxt=nxt, kn=kn):
              fire_gather(nxt, rowbs[kn], gsems[kn])
            apply(b, rowbs[k])
        return 0

      lax.fori_loop(0, (nb + NBUF - 1) // NBUF, gtri, 0)

  fire_edges(0, srcb0, dstb0, esem0)

  def cloop(c, _):
    wait_edges(c, srcb0, dstb0, esem0)
    process_chunk(c, srcb0, dstb0)
    return 0

  lax.fori_loop(0, NCH, cloop, 0)

  himask = jnp.full((L,), -65536, jnp.int32)

  def write_body(b, _):
   with jax.named_scope("wout"):
    @pl.when(n0 + b * RB < N)
    def _():
      pltpu.async_copy(
          x_hbm.at[pl.ds(n0 + b * RB, RB)],
          out_hbm.at[pl.ds(n0 + b * RB, RB), pl.ds(0, D)], lsem)
      pltpu.sync_copy(x_hbm.at[pl.ds(n0 + b * RB, RB)], inb)
      for r in range(RB):
        t = touched[b * RB + r]
        tv = lax.broadcast_in_dim(t, (L,), ()) > 0
        for j in range(D // L2):
          w = acc[b * RB + r, pl.ds(j * L, L)]
          fa = plsc.bitcast(w << 16, jnp.float32)
          fb = plsc.bitcast(w & himask, jnp.float32)
          iva = inb[r, pl.ds(j * L2, L)]
          ivb = inb[r, pl.ds(j * L2 + L, L)]
          outb[r, pl.ds(j * L2, L)] = jnp.where(tv, fa, iva)
          outb[r, pl.ds(j * L2 + L, L)] = jnp.where(tv, fb, ivb)
      pltpu.sync_copy(outb, out_hbm.at[pl.ds(n0 + b * RB, RB), pl.ds(D, D)])
    return 0

  lax.fori_loop(0, NP // RB, write_body, 0)

  def drain_left(b, _):
    @pl.when(n0 + b * RB < N)
    def _():
      pltpu.make_async_copy(
          x_hbm.at[pl.ds(n0 + b * RB, RB)],
          out_hbm.at[pl.ds(n0 + b * RB, RB), pl.ds(0, D)], lsem).wait()
    return 0

  lax.fori_loop(0, NP // RB, drain_left, 0)


@functools.partial(jax.jit, donate_argnums=())
def _gcn(x, src, dst):
  xp = (x.reshape(N, D // L2, 2, L)
        .transpose(0, 1, 3, 2)
        .reshape(N, D)
        .astype(jnp.bfloat16))
  xp = jax.lax.bitcast_convert_type(xp.reshape(N, D // 2, 2), jnp.int32)
  mesh = plsc.VectorSubcoreMesh(core_axis_name="c", subcore_axis_name="s")
  run = pl.kernel(
      _gcn_body,
      compiler_params=pltpu.CompilerParams(needs_layout_passes=False),
      out_type=jax.ShapeDtypeStruct((N, 2 * D), jnp.float32),
      mesh=mesh,
      scratch_types=[
          pltpu.VMEM((NP + 1, D // 2), jnp.int32),
          pltpu.VMEM((CH,), jnp.int32),
          pltpu.VMEM((CH,), jnp.int32),
          pltpu.VMEM((CH + B,), jnp.int32),
          pltpu.VMEM((CH + B,), jnp.int32),
          pltpu.VMEM((B, D // 2), jnp.int32),
          pltpu.VMEM((B, D // 2), jnp.int32),
          pltpu.VMEM((B, D // 2), jnp.int32),
          pltpu.VMEM((RB, D), jnp.float32),
          pltpu.VMEM((RB, D), jnp.float32),
          pltpu.SMEM((NP + 1,), jnp.int32),
          pltpu.SemaphoreType.DMA,
          pltpu.SemaphoreType.DMA,
          pltpu.SemaphoreType.DMA,
          pltpu.SemaphoreType.DMA,
          pltpu.SemaphoreType.DMA,
      ],
  )
  return run(x, xp, src, dst)


def kernel(inputs, edge_index):
  return _gcn(inputs, edge_index[0], edge_index[1])
